# baseline (speedup 1.0000x reference)
import functools

import jax
import jax.numpy as jnp
from jax import lax
from jax.experimental import pallas as pl
from jax.experimental.pallas import tpu as pltpu
from jax.experimental.pallas import tpu_sc as plsc

B = 4096
D = 512
DW = D // 2
NC = 2
NS = 16
L = 16
NW = NC * NS
BPW = B // NW
CH = 64
NCH = BPW // CH

_mesh = plsc.VectorSubcoreMesh(
    core_axis_name="c", subcore_axis_name="s", num_cores=NC, num_subcores=NS
)


@functools.partial(
    pl.kernel,
    out_type=jax.ShapeDtypeStruct((NW, L), jnp.float32),
    mesh=_mesh,
    scratch_types=[
        pltpu.VMEM((BPW,), jnp.int32),
        pltpu.VMEM((2, CH, D), jnp.float32),
        pltpu.VMEM((2, CH, DW), jnp.int32),
        pltpu.VMEM((L,), jnp.float32),
        pltpu.SemaphoreType.DMA,
        pltpu.SemaphoreType.DMA,
        pltpu.SemaphoreType.DMA,
        pltpu.SemaphoreType.DMA,
    ],
)
def _center_loss_sc(x_hbm, labels_hbm, centers_hbm, out_hbm,
                    idx_v, x_v, c_v, acc_v, sx0, sx1, sc0, sc1):
    wid = lax.axis_index("s") * NC + lax.axis_index("c")
    base = wid * BPW

    sx = (sx0, sx1)
    sc = (sc0, sc1)

    def start_x(k):
        b = k % 2
        return pltpu.async_copy(
            x_hbm.at[pl.ds(base + k * CH, CH)], x_v.at[b], sx[b])

    def start_c(k):
        b = k % 2
        return pltpu.async_copy(
            centers_hbm.at[idx_v.at[pl.ds(k * CH, CH)]], c_v.at[b], sc[b])

    px = [start_x(0), start_x(1)]
    pltpu.sync_copy(labels_hbm.at[pl.ds(base, BPW)], idx_v)
    pc = [start_c(0), start_c(1)]

    accs = [jnp.zeros((L,), jnp.float32) for _ in range(4)]

    for k in range(NCH):
        b = k % 2
        px[b].wait()
        pc[b].wait()
        if k + 2 < NCH:
            px[b] = start_x(k + 2)
            pc[b] = start_c(k + 2)

        def pair_body(rr, accs, b=b):
            a0, a1, a2, a3 = accs
            r2 = rr * 2
            for dr in range(2):
                r = r2 + dr
                for j in range(DW // L):
                    x0 = x_v[b, r, pl.ds(j * L, L)]
                    x1 = x_v[b, r, pl.ds(D // 2 + j * L, L)]
                    cw = c_v[b, r, pl.ds(j * L, L)]
                    c0 = lax.bitcast_convert_type(cw << 16, jnp.float32)
                    c1 = lax.bitcast_convert_type(cw, jnp.float32)
                    d0 = x0 - c0
                    d1 = x1 - c1
                    if j % 2 == 0:
                        a0 = a0 + d0 * d0
                        a1 = a1 + d1 * d1
                    else:
                        a2 = a2 + d0 * d0
                        a3 = a3 + d1 * d1
            return a0, a1, a2, a3

        accs = lax.fori_loop(0, CH // 2, pair_body, tuple(accs))

    total = ((accs[0] + accs[1]) + (accs[2] + accs[3])) * (0.5 / B)
    acc_v[...] = total
    pltpu.sync_copy(acc_v, out_hbm.at[wid])


def _pack_rows(a):
    rnd = lax.bitcast_convert_type(a, jnp.int32) + 0x8000
    lo, hi = rnd[:, :DW], rnd[:, DW:]
    return lax.shift_right_logical(lo, 16) | (hi & -65536)


def kernel(x, labels, centers):
    partials = _center_loss_sc(
        x, labels.astype(jnp.int32), _pack_rows(centers))
    return jnp.sum(partials)


# speedup vs baseline: 1.1641x; 1.1641x over previous
import functools

import jax
import jax.numpy as jnp
from jax import lax
from jax.experimental import pallas as pl
from jax.experimental.pallas import tpu as pltpu
from jax.experimental.pallas import tpu_sc as plsc

B = 4096
D = 512
DW = D // 2
NC = 2
NS = 16
L = 16
NW = NC * NS
BPW = B // NW
CH = 32
NCH = BPW // CH

_mesh = plsc.VectorSubcoreMesh(
    core_axis_name="c", subcore_axis_name="s", num_cores=NC, num_subcores=NS
)


@functools.partial(
    pl.kernel,
    out_type=jax.ShapeDtypeStruct((NW, L), jnp.float32),
    mesh=_mesh,
    scratch_types=[
        pltpu.VMEM((BPW,), jnp.int32),
        pltpu.VMEM((2, CH, D), jnp.float32),
        pltpu.VMEM((2, CH, DW), jnp.int32),
        pltpu.VMEM((L,), jnp.float32),
        pltpu.SemaphoreType.DMA,
        pltpu.SemaphoreType.DMA,
        pltpu.SemaphoreType.DMA,
        pltpu.SemaphoreType.DMA,
    ],
)
def _center_loss_sc(x_hbm, labels_hbm, centers_hbm, out_hbm,
                    idx_v, x_v, c_v, acc_v, sx0, sx1, sc0, sc1):
    wid = lax.axis_index("s") * NC + lax.axis_index("c")
    base = wid * BPW

    sx = (sx0, sx1)
    sc = (sc0, sc1)

    def start_x(k):
        b = k % 2
        return pltpu.async_copy(
            x_hbm.at[pl.ds(base + k * CH, CH)], x_v.at[b], sx[b])

    def start_c(k):
        b = k % 2
        return pltpu.async_copy(
            centers_hbm.at[idx_v.at[pl.ds(k * CH, CH)]], c_v.at[b], sc[b])

    px = [start_x(0), start_x(1)]
    pltpu.sync_copy(labels_hbm.at[pl.ds(base, BPW)], idx_v)
    pc = [start_c(0), start_c(1)]

    accs = [jnp.zeros((L,), jnp.float32) for _ in range(4)]

    for k in range(NCH):
        b = k % 2
        px[b].wait()
        pc[b].wait()
        if k + 2 < NCH:
            px[b] = start_x(k + 2)
            pc[b] = start_c(k + 2)

        def row_body(r, accs, b=b):
            a0, a1, a2, a3 = accs
            for j in range(DW // L):
                x0 = x_v[b, r, pl.ds(j * L, L)]
                x1 = x_v[b, r, pl.ds(D // 2 + j * L, L)]
                cw = c_v[b, r, pl.ds(j * L, L)]
                c0 = lax.bitcast_convert_type(cw << 16, jnp.float32)
                c1 = lax.bitcast_convert_type(cw, jnp.float32)
                d0 = x0 - c0
                d1 = x1 - c1
                if j % 2 == 0:
                    a0 = a0 + d0 * d0
                    a1 = a1 + d1 * d1
                else:
                    a2 = a2 + d0 * d0
                    a3 = a3 + d1 * d1
            return a0, a1, a2, a3

        accs = lax.fori_loop(0, CH, row_body, tuple(accs))

    total = ((accs[0] + accs[1]) + (accs[2] + accs[3])) * (0.5 / B)
    acc_v[...] = total
    pltpu.sync_copy(acc_v, out_hbm.at[wid])


def _pack_rows(a):
    rnd = lax.bitcast_convert_type(a, jnp.int32) + 0x8000
    lo, hi = rnd[:, :DW], rnd[:, DW:]
    return lax.shift_right_logical(lo, 16) | (hi & -65536)


def kernel(x, labels, centers):
    partials = _center_loss_sc(
        x, labels.astype(jnp.int32), _pack_rows(centers))
    return jnp.sum(partials)
